# baseline (speedup 1.0000x reference)
import functools

import jax
import jax.numpy as jnp
from jax import lax
from jax.experimental import pallas as pl
from jax.experimental.pallas import tpu as pltpu
from jax.experimental.pallas import tpu_sc as plsc

_NODE_DIM = 128
_CHUNK = 100
_N_WORKERS = 32
_ROWS_PER_W = 200


def _gather_rows_sc(table, head_idx, tail_idx):
    n_rows = 2 * head_idx.size
    half_w = _N_WORKERS // 2
    mesh = plsc.VectorSubcoreMesh(core_axis_name="c", subcore_axis_name="s")

    @functools.partial(
        pl.kernel,
        out_type=jax.ShapeDtypeStruct((n_rows, _NODE_DIM), jnp.float32),
        mesh=mesh,
        scratch_types=[
            pltpu.VMEM((2, _CHUNK), jnp.int32),
            pltpu.VMEM((_ROWS_PER_W, _NODE_DIM), jnp.float32),
            pltpu.SemaphoreType.DMA,
        ],
    )
    def gather_kernel(table_hbm, head_hbm, tail_hbm, out_hbm, idx_v, rows_v, sem):
        wid = lax.axis_index("s") * 2 + lax.axis_index("c")

        @pl.when(wid < half_w)
        def _():
            pltpu.sync_copy(head_hbm.at[pl.ds(2 * wid, 2)], idx_v)

        @pl.when(wid >= half_w)
        def _():
            pltpu.sync_copy(tail_hbm.at[pl.ds(2 * (wid - half_w), 2)], idx_v)

        copies = [
            pltpu.async_copy(table_hbm.at[idx_v.at[j]],
                             rows_v.at[pl.ds(j * _CHUNK, _CHUNK)], sem)
            for j in range(2)
        ]
        for cp in copies:
            cp.wait()
        pltpu.sync_copy(rows_v,
                        out_hbm.at[pl.ds(wid * _ROWS_PER_W, _ROWS_PER_W)])

    return gather_kernel(table, head_idx, tail_idx)


def _predict_tc(rows, W_fc, b_fc2d, W_pred, b_pred2d, B, P):
    n_pairs = rows.shape[0] // 2
    d = W_fc.shape[1]
    out_num = b_pred2d.shape[1]

    def body(rows_ref, wfc_ref, bfc_ref, wp_ref, bp_ref, out_ref):
        g = jnp.dot(rows_ref[...], wfc_ref[...],
                    preferred_element_type=jnp.float32)
        g = jnp.maximum(g + bfc_ref[...], 0.0)
        wp = wp_ref[...]
        res = (
            jnp.dot(g[:n_pairs], wp[:d], preferred_element_type=jnp.float32)
            + jnp.dot(g[n_pairs:], wp[d:], preferred_element_type=jnp.float32)
            + bp_ref[...]
        )
        out_ref[...] = res.reshape(B, P, out_num)

    return pl.pallas_call(
        body,
        out_shape=jax.ShapeDtypeStruct((B, P, out_num), jnp.float32),
    )(rows, W_fc, b_fc2d, W_pred, b_pred2d)


def kernel(node_features, edge_index, edge_features, head_ent_nodes,
           tail_ent_nodes, W_fc, b_fc, W_pred, b_pred):
    del edge_index, edge_features
    B, P = head_ent_nodes.shape
    out_num = b_pred.shape[0]
    node_dim = W_fc.shape[1]

    rows = _gather_rows_sc(node_features, head_ent_nodes, tail_ent_nodes)
    return _predict_tc(rows, W_fc, b_fc.reshape(1, node_dim),
                       W_pred, b_pred.reshape(1, out_num), B, P)


# speedup vs baseline: 1.0053x; 1.0053x over previous
import functools

import jax
import jax.numpy as jnp
from jax import lax
from jax.experimental import pallas as pl
from jax.experimental.pallas import tpu as pltpu
from jax.experimental.pallas import tpu_sc as plsc

_NODE_DIM = 128
_CHUNK = 100
_N_WORKERS = 32
_ROWS_PER_W = 200


def _gather_rows_sc(table, head_idx, tail_idx):
    n_rows = 2 * head_idx.size
    half_w = _N_WORKERS // 2
    mesh = plsc.VectorSubcoreMesh(core_axis_name="c", subcore_axis_name="s")

    @functools.partial(
        pl.kernel,
        out_type=jax.ShapeDtypeStruct((n_rows, _NODE_DIM), jnp.float32),
        mesh=mesh,
        scratch_types=[
            pltpu.VMEM((2, _CHUNK), jnp.int32),
            pltpu.VMEM((_ROWS_PER_W, _NODE_DIM), jnp.float32),
            pltpu.SemaphoreType.DMA,
        ],
    )
    def gather_kernel(table_hbm, head_hbm, tail_hbm, out_hbm, idx_v, rows_v, sem):
        wid = lax.axis_index("s") * 2 + lax.axis_index("c")

        @pl.when(wid < half_w)
        def _():
            pltpu.sync_copy(head_hbm.at[pl.ds(2 * wid, 2)], idx_v)

        @pl.when(wid >= half_w)
        def _():
            pltpu.sync_copy(tail_hbm.at[pl.ds(2 * (wid - half_w), 2)], idx_v)

        copies = [
            pltpu.async_copy(table_hbm.at[idx_v.at[j]],
                             rows_v.at[pl.ds(j * _CHUNK, _CHUNK)], sem)
            for j in range(2)
        ]
        for cp in copies:
            cp.wait()
        pltpu.sync_copy(rows_v,
                        out_hbm.at[pl.ds(wid * _ROWS_PER_W, _ROWS_PER_W)])

    return gather_kernel(table, head_idx, tail_idx)


def _predict_tc(rows, W_fc, b_fc2d, W_pred, b_pred2d, B, P):
    n_pairs = rows.shape[0] // 2
    d = W_fc.shape[1]
    out_num = b_pred2d.shape[1]

    bf = jnp.bfloat16

    def body(rows_ref, wfc_ref, bfc_ref, wp_ref, bp_ref, out_ref):
        g = jnp.dot(rows_ref[...].astype(bf), wfc_ref[...].astype(bf),
                    preferred_element_type=jnp.float32)
        g = jnp.maximum(g + bfc_ref[...], 0.0).astype(bf)
        wp = wp_ref[...].astype(bf)
        res = (
            jnp.dot(g[:n_pairs], wp[:d], preferred_element_type=jnp.float32)
            + jnp.dot(g[n_pairs:], wp[d:], preferred_element_type=jnp.float32)
            + bp_ref[...]
        )
        out_ref[...] = res.reshape(B, P, out_num)

    return pl.pallas_call(
        body,
        out_shape=jax.ShapeDtypeStruct((B, P, out_num), jnp.float32),
    )(rows, W_fc, b_fc2d, W_pred, b_pred2d)


def kernel(node_features, edge_index, edge_features, head_ent_nodes,
           tail_ent_nodes, W_fc, b_fc, W_pred, b_pred):
    del edge_index, edge_features
    B, P = head_ent_nodes.shape
    out_num = b_pred.shape[0]
    node_dim = W_fc.shape[1]

    rows = _gather_rows_sc(node_features, head_ent_nodes, tail_ent_nodes)
    return _predict_tc(rows, W_fc, b_fc.reshape(1, node_dim),
                       W_pred, b_pred.reshape(1, out_num), B, P)
